# baseline (speedup 1.0000x reference)
import jax
import jax.numpy as jnp
from jax.experimental import pallas as pl


_N_BLK = 2048


def _nnd_body(lhs_ref, rhs_ref, d1_ref, d2_ref):
    nb = pl.program_id(1)
    lhs = lhs_ref[0]
    rhs = rhs_ref[0]

    d = jax.lax.dot_general(lhs, rhs, (((1,), (0,)), ((), ())),
                            preferred_element_type=jnp.float32)

    d1_ref[0] = jnp.maximum(jnp.min(d, axis=1, keepdims=True), 0.0)

    cur = jnp.maximum(jnp.min(d, axis=0, keepdims=True), 0.0)

    @pl.when(nb == 0)
    def _init():
        d2_ref[0] = cur

    @pl.when(nb != 0)
    def _accum():
        d2_ref[0] = jnp.minimum(d2_ref[0], cur)


def kernel(input1, input2):
    B, N, _ = input1.shape
    M = input2.shape[1]
    bf16, f32 = jnp.bfloat16, jnp.float32

    x = input1
    yt = jnp.transpose(input2, (0, 2, 1))

    x2 = jnp.sum(x * x, axis=2, keepdims=True)
    y2 = jnp.sum(yt * yt, axis=1, keepdims=True)
    xm = ((-2.0) * x).astype(bf16)
    yb = yt.astype(bf16)
    x2h = x2.astype(bf16)
    x2l = (x2 - x2h.astype(f32)).astype(bf16)
    y2h = y2.astype(bf16)
    y2l = (y2 - y2h.astype(f32)).astype(bf16)
    ones_c = jnp.ones_like(x2h)
    ones_r = jnp.ones_like(y2h)
    lhs = jnp.concatenate(
        [xm, x2h, x2l, ones_c, ones_c, jnp.zeros_like(x2h)], axis=2)
    rhs = jnp.concatenate(
        [yb, ones_r, ones_r, y2h, y2l, jnp.zeros_like(y2h)], axis=1)

    nb = N // _N_BLK
    out1, out2 = pl.pallas_call(
        _nnd_body,
        grid=(B, nb),
        in_specs=[
            pl.BlockSpec((1, _N_BLK, 8), lambda b, i: (b, i, 0)),
            pl.BlockSpec((1, 8, M), lambda b, i: (b, 0, 0)),
        ],
        out_specs=[
            pl.BlockSpec((1, _N_BLK, 1), lambda b, i: (b, i, 0)),
            pl.BlockSpec((1, 1, M), lambda b, i: (b, 0, 0)),
        ],
        out_shape=[
            jax.ShapeDtypeStruct((B, N, 1), jnp.float32),
            jax.ShapeDtypeStruct((B, 1, M), jnp.float32),
        ],
    )(lhs, rhs)
    return out1.reshape(B, N), out2.reshape(B, M)
